# baseline (speedup 1.0000x reference)
import functools

import jax
import jax.numpy as jnp
from jax import lax
from jax.experimental import pallas as pl
from jax.experimental.pallas import tpu as pltpu
from jax.experimental.pallas import tpu_sc as plsc

N_ROWS = 16384
N_COLS = 5
L = 16
N_SUB = 16
CHUNK = N_ROWS * N_COLS // N_SUB
VECS = CHUNK // L
VECS_PER_CLASS = VECS // N_COLS

_mesh = plsc.VectorSubcoreMesh(core_axis_name="c", subcore_axis_name="s")


@functools.partial(
    pl.kernel,
    mesh=_mesh,
    compiler_params=pltpu.CompilerParams(needs_layout_passes=False),
    out_type=[
        jax.ShapeDtypeStruct((L,), jnp.float32),
        jax.ShapeDtypeStruct((L,), jnp.int32),
    ],
    scratch_types=[
        pltpu.VMEM((CHUNK,), jnp.float32),
        pltpu.VMEM((N_COLS * L,), jnp.float32),
        pltpu.VMEM_SHARED((N_SUB * N_COLS * L,), jnp.float32),
        pltpu.VMEM((N_SUB * N_COLS * L,), jnp.float32),
        pltpu.VMEM((N_COLS * L,), jnp.float32),
        pltpu.VMEM((L,), jnp.float32),
        pltpu.VMEM((L,), jnp.float32),
        pltpu.VMEM((L,), jnp.int32),
    ],
)
def _sc_topk(x_hbm, wt_hbm, bias_hbm, vals_hbm, idx_hbm,
             xv, accv, sharedv, gatherv, wtv, biasv, outv, outi):
    sid = lax.axis_index("s")
    cid = lax.axis_index("c")

    pltpu.sync_copy(x_hbm.at[pl.ds(sid * CHUNK, CHUNK)], xv)

    for r in range(N_COLS):
        acc = xv[pl.ds(r * L, L)]
        for j in range(1, VECS_PER_CLASS):
            acc = acc + xv[pl.ds((N_COLS * j + r) * L, L)]
        accv[pl.ds(r * L, L)] = acc

    pltpu.sync_copy(accv, sharedv.at[pl.ds(sid * (N_COLS * L), N_COLS * L)])
    plsc.subcore_barrier()

    @pl.when(sid == 0)
    def _finalize():
        pltpu.sync_copy(sharedv, gatherv)
        pltpu.sync_copy(wt_hbm, wtv)
        pltpu.sync_copy(bias_hbm, biasv)

        lanes = lax.iota(jnp.int32, L)
        zeros = jnp.zeros((L,), jnp.float32)

        cls = []
        for r in range(N_COLS):
            acc = gatherv[pl.ds(r * L, L)]
            for t in range(1, N_SUB):
                acc = acc + gatherv[pl.ds((t * N_COLS + r) * L, L)]
            cls.append(acc)

        def permute(x, idx):
            return jnp.take_along_axis(x, idx, axis=0,
                                       mode="promise_in_bounds")

        def shift_down(x, k):
            g = permute(x, lax.rem(lanes + k, L))
            return jnp.where(lanes < (L - k), g, zeros)

        colsum_vec = zeros
        for r in range(N_COLS):
            t = cls[r] + shift_down(cls[r], 5)
            t = t + shift_down(t, 10)
            colsum_vec = colsum_vec + permute(
                t, lax.rem(lanes + (N_COLS - r), N_COLS))

        scores = biasv[...] * jnp.float32(N_ROWS)
        for i in range(N_COLS):
            bcast = permute(colsum_vec, jnp.full((L,), i, jnp.int32))
            scores = scores + bcast * wtv[pl.ds(i * L, L)]

        scores = jnp.where(lanes < N_COLS, scores, jnp.float32(float("-inf")))
        skeys, svals = plsc.sort_key_val(scores, lanes, descending=True)
        outv[...] = skeys
        outi[...] = svals

        @pl.when(cid == 0)
        def _write():
            pltpu.sync_copy(outv, vals_hbm)
            pltpu.sync_copy(outi, idx_hbm)


def kernel(in_values, weight, bias, topk):
    x = in_values.reshape(-1).astype(jnp.float32)
    wt = jnp.zeros((N_COLS, L), jnp.float32).at[:, :N_COLS].set(weight.T)
    bias_pad = jnp.zeros((L,), jnp.float32).at[:N_COLS].set(bias)
    vals16, idx16 = _sc_topk(x, wt.reshape(-1), bias_pad)
    values = vals16[:4]
    indices = idx16[:4] + jnp.asarray(topk - 4, jnp.int32)
    return values, indices


# speedup vs baseline: 1.0495x; 1.0495x over previous
import functools

import jax
import jax.numpy as jnp
from jax import lax
from jax.experimental import pallas as pl
from jax.experimental.pallas import tpu as pltpu
from jax.experimental.pallas import tpu_sc as plsc

N_ROWS = 16384
N_COLS = 5
L = 16
N_SUB = 16
ROWS_PER_SUB = N_ROWS // N_SUB
CHUNK_ROWS = 512
N_CHUNKS = ROWS_PER_SUB // CHUNK_ROWS
GROUPS = CHUNK_ROWS // L

_mesh = plsc.VectorSubcoreMesh(core_axis_name="c", subcore_axis_name="s")


@functools.partial(
    pl.kernel,
    mesh=_mesh,
    compiler_params=pltpu.CompilerParams(needs_layout_passes=False),
    out_type=[
        jax.ShapeDtypeStruct((L,), jnp.float32),
        jax.ShapeDtypeStruct((L,), jnp.int32),
    ],
    scratch_types=[
        pltpu.VMEM((CHUNK_ROWS, N_COLS), jnp.float32),
        pltpu.VMEM((N_COLS * L,), jnp.float32),
        pltpu.VMEM_SHARED((N_SUB * N_COLS * L,), jnp.float32),
        pltpu.VMEM((N_SUB * N_COLS * L,), jnp.float32),
        pltpu.VMEM((N_COLS, N_COLS), jnp.float32),
        pltpu.VMEM((N_COLS,), jnp.float32),
        pltpu.VMEM((L,), jnp.float32),
        pltpu.VMEM((L,), jnp.int32),
    ],
)
def _sc_topk(x_hbm, w_hbm, b_hbm, vals_hbm, idx_hbm,
             xv, accv, sharedv, gatherv, wv, bv, outv, outi):
    sid = lax.axis_index("s")
    cid = lax.axis_index("c")

    lanes = lax.iota(jnp.int32, L)
    zeros = jnp.zeros((L,), jnp.float32)

    acc = [zeros] * N_COLS
    for chunk in range(N_CHUNKS):
        base = sid * ROWS_PER_SUB + chunk * CHUNK_ROWS
        pltpu.sync_copy(x_hbm.at[pl.ds(base, CHUNK_ROWS)], xv)
        for g in range(GROUPS):
            ridx = lanes + (g * L)
            for c in range(N_COLS):
                acc[c] = acc[c] + plsc.load_gather(
                    xv, [ridx, jnp.full((L,), c, jnp.int32)])

    for c in range(N_COLS):
        accv[pl.ds(c * L, L)] = acc[c]

    pltpu.sync_copy(accv, sharedv.at[pl.ds(sid * (N_COLS * L), N_COLS * L)])
    plsc.subcore_barrier()

    @pl.when(sid == 0)
    def _finalize():
        pltpu.sync_copy(sharedv, gatherv)
        pltpu.sync_copy(w_hbm, wv)
        pltpu.sync_copy(b_hbm, bv)

        def permute(x, idx):
            return jnp.take_along_axis(x, idx, axis=0,
                                       mode="promise_in_bounds")

        def shift_down(x, k):
            g = permute(x, lax.rem(lanes + k, L))
            return jnp.where(lanes < (L - k), g, zeros)

        lane_mod = lax.rem(lanes, N_COLS)

        bias_vec = plsc.load_gather(bv, [lane_mod])
        scores = bias_vec * jnp.float32(N_ROWS)
        for c in range(N_COLS):
            total = gatherv[pl.ds(c * L, L)]
            for t in range(1, N_SUB):
                total = total + gatherv[pl.ds((t * N_COLS + c) * L, L)]
            for k in (8, 4, 2, 1):
                total = total + shift_down(total, k)
            bcast = permute(total, jnp.zeros((L,), jnp.int32))
            wt_c = plsc.load_gather(wv, [lane_mod, jnp.full((L,), c, jnp.int32)])
            scores = scores + bcast * wt_c

        scores = jnp.where(lanes < N_COLS, scores, jnp.float32(float("-inf")))
        skeys, svals = plsc.sort_key_val(scores, lanes, descending=True)
        outv[...] = skeys
        outi[...] = svals

        @pl.when(cid == 0)
        def _write():
            pltpu.sync_copy(outv, vals_hbm)
            pltpu.sync_copy(outi, idx_hbm)


def kernel(in_values, weight, bias, topk):
    vals16, idx16 = _sc_topk(in_values, weight, bias)
    values = vals16[:4]
    indices = idx16[:4] + jnp.asarray(topk - 4, jnp.int32)
    return values, indices


# speedup vs baseline: 1.0733x; 1.0227x over previous
import functools

import jax
import jax.numpy as jnp
from jax import lax
from jax.experimental import pallas as pl
from jax.experimental.pallas import tpu as pltpu
from jax.experimental.pallas import tpu_sc as plsc

N_ROWS = 16384
N_COLS = 5
L = 16
BLOCK_ROWS = 1024
GRID = N_ROWS // BLOCK_ROWS

_NEG_INF = float("-inf")


def _tc_scores_body(x_ref, w_ref, b_ref, out_ref, acc_ref):
    i = pl.program_id(0)

    @pl.when(i == 0)
    def _init():
        acc_ref[...] = jnp.zeros_like(acc_ref)

    acc_ref[...] += jnp.sum(x_ref[...], axis=0, keepdims=True)

    @pl.when(i == GRID - 1)
    def _emit():
        colsum_t = jnp.transpose(acc_ref[...])
        wt = jnp.transpose(w_ref[...])
        scores = jnp.sum(wt * colsum_t, axis=0, keepdims=True)
        scores = scores + jnp.float32(N_ROWS) * b_ref[...]
        out_ref[...] = jnp.concatenate(
            [scores, jnp.full((1, L - N_COLS), _NEG_INF, jnp.float32)], axis=1)


_tc_scores = pl.pallas_call(
    _tc_scores_body,
    grid=(GRID,),
    in_specs=[
        pl.BlockSpec((BLOCK_ROWS, N_COLS), lambda i: (i, 0)),
        pl.BlockSpec((N_COLS, N_COLS), lambda i: (0, 0)),
        pl.BlockSpec((1, N_COLS), lambda i: (0, 0)),
    ],
    out_specs=pl.BlockSpec((1, L), lambda i: (0, 0)),
    out_shape=jax.ShapeDtypeStruct((1, L), jnp.float32),
    scratch_shapes=[pltpu.VMEM((1, N_COLS), jnp.float32)],
)

_mesh = plsc.VectorSubcoreMesh(core_axis_name="c", subcore_axis_name="s")


@functools.partial(
    pl.kernel,
    mesh=_mesh,
    compiler_params=pltpu.CompilerParams(needs_layout_passes=False),
    out_type=[
        jax.ShapeDtypeStruct((L,), jnp.float32),
        jax.ShapeDtypeStruct((L,), jnp.int32),
    ],
    scratch_types=[
        pltpu.VMEM((L,), jnp.float32),
        pltpu.VMEM((L,), jnp.float32),
        pltpu.VMEM((L,), jnp.int32),
    ],
)
def _sc_top4(scores_hbm, vals_hbm, idx_hbm, sv, ov, oi):
    sid = lax.axis_index("s")
    cid = lax.axis_index("c")

    @pl.when(jnp.logical_and(sid == 0, cid == 0))
    def _select():
        pltpu.sync_copy(scores_hbm, sv)
        lanes = lax.iota(jnp.int32, L)
        skeys, svals = plsc.sort_key_val(sv[...], lanes, descending=True)
        ov[...] = skeys
        oi[...] = svals
        pltpu.sync_copy(ov, vals_hbm)
        pltpu.sync_copy(oi, idx_hbm)


def kernel(in_values, weight, bias, topk):
    scores16 = _tc_scores(in_values, weight, bias.reshape(1, N_COLS))
    vals16, idx16 = _sc_top4(scores16.reshape(L))
    values = vals16[:4]
    indices = idx16[:4] + jnp.asarray(topk - 4, jnp.int32)
    return values, indices


# speedup vs baseline: 1.2304x; 1.1463x over previous
import functools

import jax
import jax.numpy as jnp
from jax import lax
from jax.experimental import pallas as pl
from jax.experimental.pallas import tpu as pltpu
from jax.experimental.pallas import tpu_sc as plsc

N_ROWS = 16384
N_COLS = 5
L = 16
BLOCK_ROWS = 4096
GRID = N_ROWS // BLOCK_ROWS

_NEG_INF = float("-inf")


def _tc_scores_body(x_ref, w_ref, b_ref, out_ref, acc_ref):
    i = pl.program_id(0)

    @pl.when(i == 0)
    def _init():
        acc_ref[...] = jnp.zeros_like(acc_ref)

    acc_ref[...] += jnp.sum(
        x_ref[...].reshape(BLOCK_ROWS // 8, 8, N_COLS), axis=0)

    @pl.when(i == GRID - 1)
    def _emit():
        colsum = jnp.sum(acc_ref[...], axis=0, keepdims=True)
        colsum_t = jnp.transpose(colsum)
        wt = jnp.transpose(w_ref[...])
        scores = jnp.sum(wt * colsum_t, axis=0, keepdims=True)
        scores = scores + jnp.float32(N_ROWS) * b_ref[...]
        out_ref[...] = jnp.concatenate(
            [scores, jnp.full((1, L - N_COLS), _NEG_INF, jnp.float32)], axis=1)


_tc_scores = pl.pallas_call(
    _tc_scores_body,
    grid=(GRID,),
    in_specs=[
        pl.BlockSpec((BLOCK_ROWS, N_COLS), lambda i: (i, 0)),
        pl.BlockSpec((N_COLS, N_COLS), lambda i: (0, 0)),
        pl.BlockSpec((1, N_COLS), lambda i: (0, 0)),
    ],
    out_specs=pl.BlockSpec((1, L), lambda i: (0, 0)),
    out_shape=jax.ShapeDtypeStruct((1, L), jnp.float32),
    scratch_shapes=[pltpu.VMEM((8, N_COLS), jnp.float32)],
)

_mesh = plsc.VectorSubcoreMesh(core_axis_name="c", subcore_axis_name="s")


@functools.partial(
    pl.kernel,
    mesh=_mesh,
    compiler_params=pltpu.CompilerParams(needs_layout_passes=False),
    out_type=[
        jax.ShapeDtypeStruct((L,), jnp.float32),
        jax.ShapeDtypeStruct((L,), jnp.int32),
    ],
    scratch_types=[
        pltpu.VMEM((L,), jnp.float32),
        pltpu.VMEM((L,), jnp.float32),
        pltpu.VMEM((L,), jnp.int32),
    ],
)
def _sc_top4(scores_hbm, vals_hbm, idx_hbm, sv, ov, oi):
    sid = lax.axis_index("s")
    cid = lax.axis_index("c")

    @pl.when(jnp.logical_and(sid == 0, cid == 0))
    def _select():
        pltpu.sync_copy(scores_hbm, sv)
        lanes = lax.iota(jnp.int32, L)
        skeys, svals = plsc.sort_key_val(sv[...], lanes, descending=True)
        ov[...] = skeys
        oi[...] = svals
        pltpu.sync_copy(ov, vals_hbm)
        pltpu.sync_copy(oi, idx_hbm)


def kernel(in_values, weight, bias, topk):
    scores16 = _tc_scores(in_values, weight, bias.reshape(1, N_COLS))
    vals16, idx16 = _sc_top4(scores16.reshape(L))
    values = vals16[:4]
    indices = idx16[:4] + jnp.asarray(topk - 4, jnp.int32)
    return values, indices
